# baseline (speedup 1.0000x reference)
import jax
import jax.numpy as jnp
from jax import lax
from jax.experimental import pallas as pl
from jax.experimental.pallas import tpu as pltpu
from jax.experimental.pallas import tpu_sc as plsc

N_NODES = 10000
N_EDGES = 320000
FEAT = 128
CLS = 16
NUM_GRAPHS = 64
NS = 16
NC = 2
NW = NC * NS
RPT = 625
WIN = 632
CHUNK = 128
NCH = 84
NRING = 4
EPAD = NW * NCH * CHUNK
NNCH = 3
BATCH_PAD = 10112
CROWS = 5008
CRPT = 313
CWIN = 320

_SC_PARAMS = pltpu.CompilerParams(needs_layout_passes=False)


def _aligned_win(s):
    return pl.multiple_of(s * RPT - lax.rem(s, 8), 8)


def _mm(x, w):
    bm = 1000

    def body(x_ref, w_ref, o_ref):
        o_ref[...] = jnp.dot(x_ref[...], w_ref[...],
                             preferred_element_type=jnp.float32)

    return pl.pallas_call(
        body,
        grid=(N_NODES // bm,),
        in_specs=[pl.BlockSpec((bm, FEAT), lambda i: (i, 0)),
                  pl.BlockSpec((FEAT, FEAT), lambda i: (0, 0))],
        out_specs=pl.BlockSpec((bm, FEAT), lambda i: (i, 0)),
        out_shape=jax.ShapeDtypeStruct((N_NODES, FEAT), jnp.float32),
    )(x, w)


def _fuse_relu_mm(y, acc, b2d, w):
    bm = 1000

    def body(y_ref, a_ref, b_ref, w_ref, o_ref):
        h = jnp.maximum(y_ref[...] + a_ref[0] + a_ref[1] + b_ref[...], 0.0)
        o_ref[...] = jnp.dot(h, w_ref[...], preferred_element_type=jnp.float32)

    return pl.pallas_call(
        body,
        grid=(N_NODES // bm,),
        in_specs=[pl.BlockSpec((bm, FEAT), lambda i: (i, 0)),
                  pl.BlockSpec((NC, bm, FEAT), lambda i: (0, i, 0)),
                  pl.BlockSpec((1, FEAT), lambda i: (0, 0)),
                  pl.BlockSpec((FEAT, FEAT), lambda i: (0, 0))],
        out_specs=pl.BlockSpec((bm, FEAT), lambda i: (i, 0)),
        out_shape=jax.ShapeDtypeStruct((N_NODES, FEAT), jnp.float32),
    )(y, acc, b2d, w)


def _fuse_relu(y, acc, b2d):
    bm = 1000

    def body(y_ref, a_ref, b_ref, o_ref):
        o_ref[...] = jnp.maximum(y_ref[...] + a_ref[0] + a_ref[1] + b_ref[...],
                                 0.0)

    return pl.pallas_call(
        body,
        grid=(N_NODES // bm,),
        in_specs=[pl.BlockSpec((bm, FEAT), lambda i: (i, 0)),
                  pl.BlockSpec((NC, bm, FEAT), lambda i: (0, i, 0)),
                  pl.BlockSpec((1, FEAT), lambda i: (0, 0))],
        out_specs=pl.BlockSpec((bm, FEAT), lambda i: (i, 0)),
        out_shape=jax.ShapeDtypeStruct((N_NODES, FEAT), jnp.float32),
    )(y, acc, b2d)


def _final_mm(c2, h2p, batch2d, w3, b3):
    hr = N_NODES // 2

    def body(c_ref, h_ref, be_ref, w_ref, b_ref, o_ref):
        cs = c_ref[0, :hr] + c_ref[1, :hr]
        hp = h_ref[...]
        dn = (((0,), (0,)), ((), ()))
        s = (lax.dot_general(cs[:, :NUM_GRAPHS], hp[:, :FEAT], dn,
                             preferred_element_type=jnp.float32)
             + lax.dot_general(cs[:, NUM_GRAPHS:], hp[:, FEAT:], dn,
                               preferred_element_type=jnp.float32))
        be = be_ref[...]
        cnt = jnp.stack([jnp.sum((be == g).astype(jnp.float32))
                         for g in range(NUM_GRAPHS)])
        o_ref[...] = (jnp.dot(s, w_ref[...],
                              preferred_element_type=jnp.float32)
                      + cnt[:, None] * b_ref[...][0][None, :])

    return pl.pallas_call(
        body,
        grid=(1,),
        in_specs=[pl.BlockSpec((NC, CROWS, FEAT), lambda i: (0, 0, 0)),
                  pl.BlockSpec((hr, 2 * FEAT), lambda i: (0, 0)),
                  pl.BlockSpec((BATCH_PAD // 128, 128), lambda i: (0, 0)),
                  pl.BlockSpec((FEAT, CLS), lambda i: (0, 0)),
                  pl.BlockSpec((1, CLS), lambda i: (0, 0))],
        out_specs=pl.BlockSpec((NUM_GRAPHS, CLS), lambda i: (0, 0)),
        out_shape=jax.ShapeDtypeStruct((NUM_GRAPHS, CLS), jnp.float32),
    )(c2, h2p, batch2d, w3, b3)


def _edge_acc(y, src2d, dst2d, wflat):
    mesh = plsc.VectorSubcoreMesh(core_axis_name="c", subcore_axis_name="s")

    def body(y_hbm, src_hbm, dst_hbm, w_hbm, out_hbm,
             acc_sh, src_ring, dst_ring, w_ring, rows0, rows1, rows2,
             g0, g1, g2, s0, s1, s2, i0, i1, i2, i3):
        c = lax.axis_index("c")
        s = lax.axis_index("s")
        wid = c * NS + s
        off = _aligned_win(s)
        rows = [rows0, rows1, rows2]
        gsem = [g0, g1, g2]
        ssem = [s0, s1, s2]
        isem = [i0, i1, i2, i3]

        def _triple(k, slot, fn):
            a = fn(src_hbm.at[wid, k], src_ring.at[slot], isem[slot])
            b_ = fn(dst_hbm.at[wid, k], dst_ring.at[slot], isem[slot])
            d = fn(w_hbm.at[wid, pl.ds(k * CHUNK, CHUNK)],
                   w_ring.at[pl.ds(slot * CHUNK, CHUNK)], isem[slot])
            return a, b_, d

        def issue_triple(k, slot):
            _triple(k, slot, pltpu.async_copy)

        def wait_triple(k, slot):
            for d in _triple(k, slot, pltpu.make_async_copy):
                d.wait()

        zv = jnp.zeros((16,), jnp.float32)

        def zrow(e, cc):
            for f in range(FEAT // 16):
                rows0[e, pl.ds(16 * f, 16)] = zv
            return cc

        lax.fori_loop(0, CHUNK, zrow, 0)
        for jo in range(5):
            sz = 128 if jo < 4 else WIN - 512
            base = pl.multiple_of(off + jo * 128, 8)
            pltpu.sync_copy(rows0.at[pl.ds(0, sz)],
                            acc_sh.at[pl.ds(base, sz)])
        plsc.subcore_barrier()

        def _scale(buf, slot):
            def group_body(g, c2):
                base16 = jnp.full((16,), slot * CHUNK + g * 16, jnp.int32)
                for lane in range(16):
                    kv = base16 + lane
                    wv = plsc.load_gather(w_ring, [kv])
                    e = g * 16 + lane
                    for f in range(FEAT // 16):
                        sl = pl.ds(16 * f, 16)
                        buf[e, sl] = buf[e, sl] * wv
                return c2

            lax.fori_loop(0, CHUNK // 16, group_body, 0)

        for k in range(3):
            issue_triple(k, k)
        for b0 in range(2):
            wait_triple(b0, b0)
            pltpu.async_copy(y_hbm.at[src_ring.at[b0]], rows[b0], gsem[b0])

        @pl.loop(0, NCH, step=12)
        def _pipe(j):
            for u in range(12):
                jb = j + u
                b = u % 3
                r = u % NRING
                pltpu.make_async_copy(y_hbm.at[src_ring.at[r]],
                                      rows[b], gsem[b]).wait()

                @pl.when(jb >= 1)
                def _drain(b=b, r=r):
                    pltpu.make_async_copy(
                        rows[(b + 2) % 3],
                        acc_sh.at[dst_ring.at[(r + 3) % NRING]],
                        ssem[(b + 2) % 3]).wait()

                @pl.when(jb < NCH - 2)
                def _gather(jb=jb, b=b, r=r):
                    wait_triple(jb + 2, (r + 2) % NRING)
                    pltpu.async_copy(
                        y_hbm.at[src_ring.at[(r + 2) % NRING]],
                        rows[(b + 2) % 3], gsem[(b + 2) % 3])

                @pl.when(jb < NCH - 3)
                def _prefetch(jb=jb, r=r):
                    issue_triple(jb + 3, (r + 3) % NRING)

                _scale(rows[b], r)
                pltpu.async_copy(rows[b], acc_sh.at[dst_ring.at[r]],
                                 ssem[b], add=True)

        pltpu.make_async_copy(rows[2], acc_sh.at[dst_ring.at[NRING - 1]],
                              ssem[2]).wait()
        plsc.subcore_barrier()

        for jo in range(5):
            sz = 128 if jo < 4 else WIN - 512
            base = pl.multiple_of(off + jo * 128, 8)
            pltpu.sync_copy(acc_sh.at[pl.ds(base, sz)],
                            out_hbm.at[c, pl.ds(base, sz)])

    kern = pl.kernel(
        body,
        out_type=jax.ShapeDtypeStruct((NC, N_NODES, FEAT), jnp.float32),
        mesh=mesh,
        compiler_params=_SC_PARAMS,
        scratch_types=(
            [pltpu.VMEM_SHARED((N_NODES, FEAT), jnp.float32),
             pltpu.VMEM((NRING, CHUNK), jnp.int32),
             pltpu.VMEM((NRING, CHUNK), jnp.int32),
             pltpu.VMEM((NRING * CHUNK,), jnp.float32),
             pltpu.VMEM((CHUNK, FEAT), jnp.float32),
             pltpu.VMEM((CHUNK, FEAT), jnp.float32),
             pltpu.VMEM((CHUNK, FEAT), jnp.float32)]
            + [pltpu.SemaphoreType.DMA] * 10
        ),
    )
    return kern(y, src2d, dst2d, wflat)


def _build_c(srcflat, dstflat, wflat, batch_ext, nidflat):
    mesh = plsc.VectorSubcoreMesh(core_axis_name="c", subcore_axis_name="s")

    def body(src_hbm, dst_hbm, w_hbm, batch_hbm, nidf_hbm, out_hbm,
             c_sh, src_v, dst_v, w_v, batch_v, nidf_v,
             msg0, msg1, mix0, mix1, rr0, rr1, p0, p1):
        c = lax.axis_index("c")
        s = lax.axis_index("s")
        wid = c * NS + s
        off = pl.multiple_of(s * CRPT - lax.rem(s, 8), 8)
        pltpu.sync_copy(src_hbm.at[wid], src_v)
        pltpu.sync_copy(dst_hbm.at[wid], dst_v)
        pltpu.sync_copy(w_hbm.at[wid], w_v)
        pltpu.sync_copy(batch_hbm, batch_v)
        pltpu.sync_copy(nidf_hbm.at[wid], nidf_v)
        msg = [msg0, msg1]
        mix = [mix0, mix1]
        rr = [rr0, rr1]
        psem = [p0, p1]

        iota16 = lax.iota(jnp.int32, 16)
        zvf = jnp.zeros((16,), jnp.float32)
        ones_f = jnp.ones((16,), jnp.float32)

        for mb in msg:
            def zrow(e, cc, _mb=mb):
                for f in range(FEAT // 16):
                    _mb[e, pl.ds(16 * f, 16)] = zvf
                return cc

            lax.fori_loop(0, CHUNK, zrow, 0)
        for jo, sz in ((0, 128), (1, 128), (2, CWIN - 256)):
            base = pl.multiple_of(off + jo * 128, 8)
            pltpu.sync_copy(msg0.at[pl.ds(0, sz)], c_sh.at[pl.ds(base, sz)])
        plsc.subcore_barrier()

        def edge_group(b, g, jb):
            st = pl.multiple_of(jb * CHUNK + g * 16, 16)
            s16 = src_v[pl.ds(st, 16)]
            d16 = dst_v[pl.ds(st, 16)]
            w16 = w_v[pl.ds(st, 16)]
            g16 = plsc.load_gather(batch_v, [d16])
            col16 = (s16 & 1) * NUM_GRAPHS + g16
            row16 = lax.shift_right_logical(s16, 1)
            eloc16 = iota16 + g * 16
            plsc.store_scatter(msg[b], [eloc16, col16], w16)
            mix[b][pl.ds(pl.multiple_of(g * 16, 16), 16)] = col16
            rr[b][0, pl.ds(pl.multiple_of(g * 16, 16), 16)] = row16

        def zero_group(b, g):
            col16 = mix[b][pl.ds(pl.multiple_of(g * 16, 16), 16)]
            eloc16 = iota16 + g * 16
            plsc.store_scatter(msg[b], [eloc16, col16], zvf)

        @pl.loop(0, NCH, step=2)
        def _pipe(j):
            for b in range(2):
                jb = j + b

                @pl.when(jb >= 2)
                def _drain(b=b):
                    pltpu.make_async_copy(msg[b], c_sh.at[rr[b].at[0]],
                                          psem[b]).wait()

                    def zg(g, cc, _b=b):
                        zero_group(_b, g)
                        return cc

                    lax.fori_loop(0, CHUNK // 16, zg, 0)

                def bg(g, cc, _b=b, _jb=jb):
                    edge_group(_b, g, _jb)
                    return cc

                lax.fori_loop(0, CHUNK // 16, bg, 0)
                pltpu.async_copy(msg[b], c_sh.at[rr[b].at[0]], psem[b],
                                 add=True)

        for b in range(2):
            pltpu.make_async_copy(msg[b], c_sh.at[rr[b].at[0]],
                                  psem[b]).wait()

            def zg2(g, cc, _b=b):
                zero_group(_b, g)
                return cc

            lax.fori_loop(0, CHUNK // 16, zg2, 0)

        for jn in range(NNCH):
            def ng(g, cc, _jn=jn):
                st = pl.multiple_of(_jn * CHUNK + g * 16, 16)
                n16 = nidf_v[pl.ds(st, 16)]
                g16 = plsc.load_gather(batch_v, [n16])
                col16 = (n16 & 1) * NUM_GRAPHS + g16
                row16 = lax.shift_right_logical(n16, 1)
                eloc16 = iota16 + g * 16
                plsc.store_scatter(msg0, [eloc16, col16], ones_f)
                mix0[pl.ds(pl.multiple_of(g * 16, 16), 16)] = col16
                rr0[0, pl.ds(pl.multiple_of(g * 16, 16), 16)] = row16
                return cc

            lax.fori_loop(0, CHUNK // 16, ng, 0)
            pltpu.sync_copy(msg0, c_sh.at[rr0.at[0]], add=True)

            def zg3(g, cc):
                zero_group(0, g)
                return cc

            lax.fori_loop(0, CHUNK // 16, zg3, 0)

        plsc.subcore_barrier()
        for jo, sz in ((0, 128), (1, 128), (2, CWIN - 256)):
            base = pl.multiple_of(off + jo * 128, 8)
            pltpu.sync_copy(c_sh.at[pl.ds(base, sz)],
                            out_hbm.at[c, pl.ds(base, sz)])

    kern = pl.kernel(
        body,
        out_type=jax.ShapeDtypeStruct((NC, CROWS, FEAT), jnp.float32),
        mesh=mesh,
        compiler_params=_SC_PARAMS,
        scratch_types=[
            pltpu.VMEM_SHARED((CROWS, FEAT), jnp.float32),
            pltpu.VMEM((NCH * CHUNK,), jnp.int32),
            pltpu.VMEM((NCH * CHUNK,), jnp.int32),
            pltpu.VMEM((NCH * CHUNK,), jnp.float32),
            pltpu.VMEM((BATCH_PAD,), jnp.int32),
            pltpu.VMEM((NNCH * CHUNK,), jnp.int32),
            pltpu.VMEM((CHUNK, FEAT), jnp.float32),
            pltpu.VMEM((CHUNK, FEAT), jnp.float32),
            pltpu.VMEM((CHUNK,), jnp.int32),
            pltpu.VMEM((CHUNK,), jnp.int32),
            pltpu.VMEM((1, CHUNK), jnp.int32),
            pltpu.VMEM((1, CHUNK), jnp.int32),
            pltpu.SemaphoreType.DMA,
            pltpu.SemaphoreType.DMA,
        ],
    )
    return kern(srcflat, dstflat, wflat, batch_ext, nidflat)


def kernel(x, edge_index, edge_weight, batch, W1, b1, W2, b2, W3, b3):
    src = edge_index[0].astype(jnp.int32)
    dst = edge_index[1].astype(jnp.int32)
    w = edge_weight.astype(jnp.float32)

    pad = EPAD - N_EDGES
    spread = jnp.arange(pad, dtype=jnp.int32) % N_NODES
    srcp = jnp.concatenate([src, spread])
    dstp = jnp.concatenate([dst, spread])
    wp = jnp.concatenate([w, jnp.zeros((pad,), jnp.float32)])
    src2d = srcp.reshape(NW, NCH, CHUNK)
    dst2d = dstp.reshape(NW, NCH, CHUNK)
    srcflat = srcp.reshape(NW, NCH * CHUNK)
    dstflat = dstp.reshape(NW, NCH * CHUNK)
    wflat = wp.reshape(NW, NCH * CHUNK)

    batch_ext = jnp.concatenate([
        batch.astype(jnp.int32),
        jnp.full((BATCH_PAD - N_NODES,), NUM_GRAPHS, jnp.int32)])
    batch2d = batch_ext.reshape(BATCH_PAD // 128, 128)

    k = jnp.arange(NNCH * CHUNK, dtype=jnp.int32)[None, :]
    wrow = jnp.arange(NW, dtype=jnp.int32)[:, None]
    nid = wrow * CRPT + k
    nidflat = jnp.where((k < CRPT) & (nid < N_NODES), nid,
                        N_NODES + (k % 16))

    b1_2d = b1.reshape(1, FEAT)
    b2_2d = b2.reshape(1, FEAT)
    b3_2d = b3.reshape(1, CLS)

    y1 = _mm(x, W1)
    a1 = _edge_acc(y1, src2d, dst2d, wflat)
    y2 = _fuse_relu_mm(y1, a1, b1_2d, W2)
    a2 = _edge_acc(y2, src2d, dst2d, wflat)
    h2 = _fuse_relu(y2, a2, b2_2d)
    h2p = h2.reshape(N_NODES // 2, 2 * FEAT)
    c2 = _build_c(srcflat, dstflat, wflat, batch_ext, nidflat)
    return _final_mm(c2, h2p, batch2d, W3, b3_2d)


# speedup vs baseline: 1.0245x; 1.0245x over previous
import jax
import jax.numpy as jnp
from jax import lax
from jax.experimental import pallas as pl
from jax.experimental.pallas import tpu as pltpu
from jax.experimental.pallas import tpu_sc as plsc

N_NODES = 10000
N_EDGES = 320000
FEAT = 128
CLS = 16
NUM_GRAPHS = 64
NS = 16
NC = 2
NW = NC * NS
RPT = 625
WIN = 632
CHUNK = 128
NCH = 84
NBUF = 4
NRING = 4
EPAD = NW * NCH * CHUNK
NNCH = 3
BATCH_PAD = 10112
CROWS = 5008
CRPT = 313
CWIN = 320

_SC_PARAMS = pltpu.CompilerParams(needs_layout_passes=False)


def _aligned_win(s):
    return pl.multiple_of(s * RPT - lax.rem(s, 8), 8)


def _mm(x, w):
    bm = 1000

    def body(x_ref, w_ref, o_ref):
        o_ref[...] = jnp.dot(x_ref[...], w_ref[...],
                             preferred_element_type=jnp.float32)

    return pl.pallas_call(
        body,
        grid=(N_NODES // bm,),
        in_specs=[pl.BlockSpec((bm, FEAT), lambda i: (i, 0)),
                  pl.BlockSpec((FEAT, FEAT), lambda i: (0, 0))],
        out_specs=pl.BlockSpec((bm, FEAT), lambda i: (i, 0)),
        out_shape=jax.ShapeDtypeStruct((N_NODES, FEAT), jnp.float32),
    )(x, w)


def _fuse_relu_mm(y, acc, b2d, w):
    bm = 1000

    def body(y_ref, a_ref, b_ref, w_ref, o_ref):
        h = jnp.maximum(y_ref[...] + a_ref[0] + a_ref[1] + b_ref[...], 0.0)
        o_ref[...] = jnp.dot(h, w_ref[...], preferred_element_type=jnp.float32)

    return pl.pallas_call(
        body,
        grid=(N_NODES // bm,),
        in_specs=[pl.BlockSpec((bm, FEAT), lambda i: (i, 0)),
                  pl.BlockSpec((NC, bm, FEAT), lambda i: (0, i, 0)),
                  pl.BlockSpec((1, FEAT), lambda i: (0, 0)),
                  pl.BlockSpec((FEAT, FEAT), lambda i: (0, 0))],
        out_specs=pl.BlockSpec((bm, FEAT), lambda i: (i, 0)),
        out_shape=jax.ShapeDtypeStruct((N_NODES, FEAT), jnp.float32),
    )(y, acc, b2d, w)


def _fuse_relu(y, acc, b2d):
    bm = 1000

    def body(y_ref, a_ref, b_ref, o_ref):
        o_ref[...] = jnp.maximum(y_ref[...] + a_ref[0] + a_ref[1] + b_ref[...],
                                 0.0)

    return pl.pallas_call(
        body,
        grid=(N_NODES // bm,),
        in_specs=[pl.BlockSpec((bm, FEAT), lambda i: (i, 0)),
                  pl.BlockSpec((NC, bm, FEAT), lambda i: (0, i, 0)),
                  pl.BlockSpec((1, FEAT), lambda i: (0, 0))],
        out_specs=pl.BlockSpec((bm, FEAT), lambda i: (i, 0)),
        out_shape=jax.ShapeDtypeStruct((N_NODES, FEAT), jnp.float32),
    )(y, acc, b2d)


def _final_mm(c2, h2p, batch2d, w3, b3):
    hr = N_NODES // 2

    def body(c_ref, h_ref, be_ref, w_ref, b_ref, o_ref):
        cs = c_ref[0, :hr] + c_ref[1, :hr]
        hp = h_ref[...]
        dn = (((0,), (0,)), ((), ()))
        s = (lax.dot_general(cs[:, :NUM_GRAPHS], hp[:, :FEAT], dn,
                             preferred_element_type=jnp.float32)
             + lax.dot_general(cs[:, NUM_GRAPHS:], hp[:, FEAT:], dn,
                               preferred_element_type=jnp.float32))
        be = be_ref[...]
        cnt = jnp.stack([jnp.sum((be == g).astype(jnp.float32))
                         for g in range(NUM_GRAPHS)])
        o_ref[...] = (jnp.dot(s, w_ref[...],
                              preferred_element_type=jnp.float32)
                      + cnt[:, None] * b_ref[...][0][None, :])

    return pl.pallas_call(
        body,
        grid=(1,),
        in_specs=[pl.BlockSpec((NC, CROWS, FEAT), lambda i: (0, 0, 0)),
                  pl.BlockSpec((hr, 2 * FEAT), lambda i: (0, 0)),
                  pl.BlockSpec((BATCH_PAD // 128, 128), lambda i: (0, 0)),
                  pl.BlockSpec((FEAT, CLS), lambda i: (0, 0)),
                  pl.BlockSpec((1, CLS), lambda i: (0, 0))],
        out_specs=pl.BlockSpec((NUM_GRAPHS, CLS), lambda i: (0, 0)),
        out_shape=jax.ShapeDtypeStruct((NUM_GRAPHS, CLS), jnp.float32),
    )(c2, h2p, batch2d, w3, b3)


def _edge_acc(y, src2d, dst2d, wflat):
    mesh = plsc.VectorSubcoreMesh(core_axis_name="c", subcore_axis_name="s")

    def body(y_hbm, src_hbm, dst_hbm, w_hbm, out_hbm,
             acc_sh, src_ring, dst_ring, w_ring, rows0, rows1,
             g0, g1, s0, s1, i0, i1, i2, i3):
        c = lax.axis_index("c")
        s = lax.axis_index("s")
        wid = c * NS + s
        off = _aligned_win(s)
        rows = [rows0, rows1]
        gsem = [g0, g1]
        ssem = [s0, s1]
        isem = [i0, i1, i2, i3]

        def _triple(k, slot, fn):
            a = fn(src_hbm.at[wid, k], src_ring.at[slot], isem[slot])
            b_ = fn(dst_hbm.at[wid, k], dst_ring.at[slot], isem[slot])
            d = fn(w_hbm.at[wid, pl.ds(k * CHUNK, CHUNK)],
                   w_ring.at[pl.ds(slot * CHUNK, CHUNK)], isem[slot])
            return a, b_, d

        def issue_triple(k, slot):
            _triple(k, slot, pltpu.async_copy)

        def wait_triple(k, slot):
            for d in _triple(k, slot, pltpu.make_async_copy):
                d.wait()

        zv = jnp.zeros((16,), jnp.float32)

        def zrow(e, cc):
            for f in range(FEAT // 16):
                rows0[e, pl.ds(16 * f, 16)] = zv
            return cc

        lax.fori_loop(0, CHUNK, zrow, 0)
        for jo in range(5):
            sz = 128 if jo < 4 else WIN - 512
            base = pl.multiple_of(off + jo * 128, 8)
            pltpu.sync_copy(rows0.at[pl.ds(0, sz)],
                            acc_sh.at[pl.ds(base, sz)])
        plsc.subcore_barrier()

        def _scale(buf, slot):
            def group_body(g, c2):
                base16 = jnp.full((16,), slot * CHUNK + g * 16, jnp.int32)
                for lane in range(16):
                    kv = base16 + lane
                    wv = plsc.load_gather(w_ring, [kv])
                    e = g * 16 + lane
                    for f in range(FEAT // 16):
                        sl = pl.ds(16 * f, 16)
                        buf[e, sl] = buf[e, sl] * wv
                return c2

            lax.fori_loop(0, CHUNK // 16, group_body, 0)

        for k in range(3):
            issue_triple(k, k)
        wait_triple(0, 0)
        pltpu.async_copy(y_hbm.at[src_ring.at[0]], rows[0], gsem[0])

        @pl.loop(0, NCH, step=NRING)
        def _pipe(j):
            for u in range(NRING):
                jb = j + u
                b = u % 2
                r = u
                pltpu.make_async_copy(y_hbm.at[src_ring.at[r]],
                                      rows[b], gsem[b]).wait()

                @pl.when(jb >= 1)
                def _drain(b=b, r=r):
                    pltpu.make_async_copy(
                        rows[1 - b],
                        acc_sh.at[dst_ring.at[(r + 3) % NRING]],
                        ssem[1 - b]).wait()

                @pl.when(jb < NCH - 1)
                def _gather(jb=jb, b=b, r=r):
                    wait_triple(jb + 1, (r + 1) % NRING)
                    pltpu.async_copy(
                        y_hbm.at[src_ring.at[(r + 1) % NRING]],
                        rows[1 - b], gsem[1 - b])

                @pl.when(jb < NCH - 3)
                def _prefetch(jb=jb, r=r):
                    issue_triple(jb + 3, (r + 3) % NRING)

                _scale(rows[b], r)
                pltpu.async_copy(rows[b], acc_sh.at[dst_ring.at[r]],
                                 ssem[b], add=True)

        pltpu.make_async_copy(rows[1], acc_sh.at[dst_ring.at[NRING - 1]],
                              ssem[1]).wait()
        plsc.subcore_barrier()

        for jo in range(5):
            sz = 128 if jo < 4 else WIN - 512
            base = pl.multiple_of(off + jo * 128, 8)
            pltpu.sync_copy(acc_sh.at[pl.ds(base, sz)],
                            out_hbm.at[c, pl.ds(base, sz)])

    kern = pl.kernel(
        body,
        out_type=jax.ShapeDtypeStruct((NC, N_NODES, FEAT), jnp.float32),
        mesh=mesh,
        compiler_params=_SC_PARAMS,
        scratch_types=(
            [pltpu.VMEM_SHARED((N_NODES, FEAT), jnp.float32),
             pltpu.VMEM((NRING, CHUNK), jnp.int32),
             pltpu.VMEM((NRING, CHUNK), jnp.int32),
             pltpu.VMEM((NRING * CHUNK,), jnp.float32),
             pltpu.VMEM((CHUNK, FEAT), jnp.float32),
             pltpu.VMEM((CHUNK, FEAT), jnp.float32)]
            + [pltpu.SemaphoreType.DMA] * 8
        ),
    )
    return kern(y, src2d, dst2d, wflat)


def _build_c(srcflat, dstflat, wflat, batch_ext, nidflat):
    mesh = plsc.VectorSubcoreMesh(core_axis_name="c", subcore_axis_name="s")

    def body(src_hbm, dst_hbm, w_hbm, batch_hbm, nidf_hbm, out_hbm,
             c_sh, src_v, dst_v, w_v, batch_v, nidf_v,
             msg0, msg1, mix0, mix1, rr0, rr1, p0, p1):
        c = lax.axis_index("c")
        s = lax.axis_index("s")
        wid = c * NS + s
        off = pl.multiple_of(s * CRPT - lax.rem(s, 8), 8)
        pltpu.sync_copy(src_hbm.at[wid], src_v)
        pltpu.sync_copy(dst_hbm.at[wid], dst_v)
        pltpu.sync_copy(w_hbm.at[wid], w_v)
        pltpu.sync_copy(batch_hbm, batch_v)
        pltpu.sync_copy(nidf_hbm.at[wid], nidf_v)
        msg = [msg0, msg1]
        mix = [mix0, mix1]
        rr = [rr0, rr1]
        psem = [p0, p1]

        iota16 = lax.iota(jnp.int32, 16)
        zvf = jnp.zeros((16,), jnp.float32)
        ones_f = jnp.ones((16,), jnp.float32)

        for mb in msg:
            def zrow(e, cc, _mb=mb):
                for f in range(FEAT // 16):
                    _mb[e, pl.ds(16 * f, 16)] = zvf
                return cc

            lax.fori_loop(0, CHUNK, zrow, 0)
        for jo, sz in ((0, 128), (1, 128), (2, CWIN - 256)):
            base = pl.multiple_of(off + jo * 128, 8)
            pltpu.sync_copy(msg0.at[pl.ds(0, sz)], c_sh.at[pl.ds(base, sz)])
        plsc.subcore_barrier()

        def edge_group(b, g, jb):
            st = pl.multiple_of(jb * CHUNK + g * 16, 16)
            s16 = src_v[pl.ds(st, 16)]
            d16 = dst_v[pl.ds(st, 16)]
            w16 = w_v[pl.ds(st, 16)]
            g16 = plsc.load_gather(batch_v, [d16])
            col16 = (s16 & 1) * NUM_GRAPHS + g16
            row16 = lax.shift_right_logical(s16, 1)
            eloc16 = iota16 + g * 16
            plsc.store_scatter(msg[b], [eloc16, col16], w16)
            mix[b][pl.ds(pl.multiple_of(g * 16, 16), 16)] = col16
            rr[b][0, pl.ds(pl.multiple_of(g * 16, 16), 16)] = row16

        def zero_group(b, g):
            col16 = mix[b][pl.ds(pl.multiple_of(g * 16, 16), 16)]
            eloc16 = iota16 + g * 16
            plsc.store_scatter(msg[b], [eloc16, col16], zvf)

        @pl.loop(0, NCH, step=2)
        def _pipe(j):
            for b in range(2):
                jb = j + b

                @pl.when(jb >= 2)
                def _drain(b=b):
                    pltpu.make_async_copy(msg[b], c_sh.at[rr[b].at[0]],
                                          psem[b]).wait()

                    def zg(g, cc, _b=b):
                        zero_group(_b, g)
                        return cc

                    lax.fori_loop(0, CHUNK // 16, zg, 0)

                def bg(g, cc, _b=b, _jb=jb):
                    edge_group(_b, g, _jb)
                    return cc

                lax.fori_loop(0, CHUNK // 16, bg, 0)
                pltpu.async_copy(msg[b], c_sh.at[rr[b].at[0]], psem[b],
                                 add=True)

        for b in range(2):
            pltpu.make_async_copy(msg[b], c_sh.at[rr[b].at[0]],
                                  psem[b]).wait()

            def zg2(g, cc, _b=b):
                zero_group(_b, g)
                return cc

            lax.fori_loop(0, CHUNK // 16, zg2, 0)

        for jn in range(NNCH):
            def ng(g, cc, _jn=jn):
                st = pl.multiple_of(_jn * CHUNK + g * 16, 16)
                n16 = nidf_v[pl.ds(st, 16)]
                g16 = plsc.load_gather(batch_v, [n16])
                col16 = (n16 & 1) * NUM_GRAPHS + g16
                row16 = lax.shift_right_logical(n16, 1)
                eloc16 = iota16 + g * 16
                plsc.store_scatter(msg0, [eloc16, col16], ones_f)
                mix0[pl.ds(pl.multiple_of(g * 16, 16), 16)] = col16
                rr0[0, pl.ds(pl.multiple_of(g * 16, 16), 16)] = row16
                return cc

            lax.fori_loop(0, CHUNK // 16, ng, 0)
            pltpu.sync_copy(msg0, c_sh.at[rr0.at[0]], add=True)

            def zg3(g, cc):
                zero_group(0, g)
                return cc

            lax.fori_loop(0, CHUNK // 16, zg3, 0)

        plsc.subcore_barrier()
        for jo, sz in ((0, 128), (1, 128), (2, CWIN - 256)):
            base = pl.multiple_of(off + jo * 128, 8)
            pltpu.sync_copy(c_sh.at[pl.ds(base, sz)],
                            out_hbm.at[c, pl.ds(base, sz)])

    kern = pl.kernel(
        body,
        out_type=jax.ShapeDtypeStruct((NC, CROWS, FEAT), jnp.float32),
        mesh=mesh,
        compiler_params=_SC_PARAMS,
        scratch_types=[
            pltpu.VMEM_SHARED((CROWS, FEAT), jnp.float32),
            pltpu.VMEM((NCH * CHUNK,), jnp.int32),
            pltpu.VMEM((NCH * CHUNK,), jnp.int32),
            pltpu.VMEM((NCH * CHUNK,), jnp.float32),
            pltpu.VMEM((BATCH_PAD,), jnp.int32),
            pltpu.VMEM((NNCH * CHUNK,), jnp.int32),
            pltpu.VMEM((CHUNK, FEAT), jnp.float32),
            pltpu.VMEM((CHUNK, FEAT), jnp.float32),
            pltpu.VMEM((CHUNK,), jnp.int32),
            pltpu.VMEM((CHUNK,), jnp.int32),
            pltpu.VMEM((1, CHUNK), jnp.int32),
            pltpu.VMEM((1, CHUNK), jnp.int32),
            pltpu.SemaphoreType.DMA,
            pltpu.SemaphoreType.DMA,
        ],
    )
    return kern(srcflat, dstflat, wflat, batch_ext, nidflat)


def kernel(x, edge_index, edge_weight, batch, W1, b1, W2, b2, W3, b3):
    src = edge_index[0].astype(jnp.int32)
    dst = edge_index[1].astype(jnp.int32)
    w = edge_weight.astype(jnp.float32)

    pad = EPAD - N_EDGES
    spread = jnp.arange(pad, dtype=jnp.int32) % N_NODES
    srcp = jnp.concatenate([src, spread])
    dstp = jnp.concatenate([dst, spread])
    wp = jnp.concatenate([w, jnp.zeros((pad,), jnp.float32)])
    src2d = srcp.reshape(NW, NCH, CHUNK)
    dst2d = dstp.reshape(NW, NCH, CHUNK)
    srcflat = srcp.reshape(NW, NCH * CHUNK)
    dstflat = dstp.reshape(NW, NCH * CHUNK)
    wflat = wp.reshape(NW, NCH * CHUNK)

    batch_ext = jnp.concatenate([
        batch.astype(jnp.int32),
        jnp.full((BATCH_PAD - N_NODES,), NUM_GRAPHS, jnp.int32)])
    batch2d = batch_ext.reshape(BATCH_PAD // 128, 128)

    k = jnp.arange(NNCH * CHUNK, dtype=jnp.int32)[None, :]
    wrow = jnp.arange(NW, dtype=jnp.int32)[:, None]
    nid = wrow * CRPT + k
    nidflat = jnp.where((k < CRPT) & (nid < N_NODES), nid,
                        N_NODES + (k % 16))

    b1_2d = b1.reshape(1, FEAT)
    b2_2d = b2.reshape(1, FEAT)
    b3_2d = b3.reshape(1, CLS)

    y1 = _mm(x, W1)
    a1 = _edge_acc(y1, src2d, dst2d, wflat)
    y2 = _fuse_relu_mm(y1, a1, b1_2d, W2)
    a2 = _edge_acc(y2, src2d, dst2d, wflat)
    h2 = _fuse_relu(y2, a2, b2_2d)
    h2p = h2.reshape(N_NODES // 2, 2 * FEAT)
    c2 = _build_c(srcflat, dstflat, wflat, batch_ext, nidflat)
    return _final_mm(c2, h2p, batch2d, W3, b3_2d)
